# Initial kernel scaffold; baseline (speedup 1.0000x reference)
import jax
import jax.numpy as jnp
from jax.experimental import pallas as pl


def kernel(src, dst, p0, p1, p2, w1, b1, w2, b2):
    raise NotImplementedError("write your pallas kernel here")


# speedup vs baseline: 1.9882x; 1.9882x over previous
import functools

import jax
import jax.numpy as jnp
from jax import lax
from jax.experimental import pallas as pl
from jax.experimental.pallas import tpu as pltpu
from jax.experimental.pallas import tpu_sc as plsc

NUM_NODES = 100000
DIM = 150
OUT_DIM = 36
HID = 144

NC = 2
NS = 16
L = 16
NW = NC * NS

B = 100000
PER_W = 3136
BP = NW * PER_W
CHUNK = 64
N_CHUNKS = PER_W // CHUNK
N_GROUPS = CHUNK // L

SEGS = [
    (0, 84, (0, 1, 2), (0, 22, 44)),
    (84, 106, (0, 1, 3), (0, 22, -84)),
    (106, 128, (0, 2, 3), (0, -106, -84)),
    (128, 150, (1, 2, 3), (-128, -106, -84)),
]
G_COL = 66
G_PAIR = {(0, 0): 0, (0, 1): 1, (0, 2): 2, (1, 1): 3, (1, 2): 4, (2, 2): 5}


RN = 2000


def _ft_body(p0r, p1r, p2r, f0r, f1r, f2r, f3r):
    a0, a1, a2 = p0r[...], p1r[...], p2r[...]
    gs = []
    for (x, y) in [(a0, a0), (a0, a1), (a0, a2),
                   (a1, a1), (a1, a2), (a2, a2)]:
        gs.append(jnp.sum(x * y, axis=1, keepdims=True))
    f0r[...] = a0[:, :128]
    f1r[...] = jnp.concatenate([a0[:, 128:], a1[:, :106]], axis=1)
    f2r[...] = jnp.concatenate([a1[:, 106:], a2[:, :84]], axis=1)
    f3r[...] = jnp.concatenate(
        [a2[:, 84:]] + gs + [jnp.zeros((RN, 56), jnp.float32)], axis=1)


def _build_ft(p0, p1, p2):
    grid = (NUM_NODES // RN,)
    ft_shape = jax.ShapeDtypeStruct((NUM_NODES, 128), jnp.float32)
    return pl.pallas_call(
        _ft_body,
        grid=grid,
        in_specs=[pl.BlockSpec((RN, DIM), lambda i: (i, 0))] * 3,
        out_specs=[pl.BlockSpec((RN, 128), lambda i: (i, 0))] * 4,
        out_shape=[ft_shape] * 4,
    )(p0, p1, p2)


def _sc_body(src_h, dst_h, f0, f1, f2, f3, rf_h,
             idx_s, idx_d, s0, s1, s2, s3, d0, d1, d2, d3, rfb, sem):
    cid = lax.axis_index("c")
    sid = lax.axis_index("s")
    wid = sid * NC + cid
    base = wid * PER_W

    pltpu.sync_copy(src_h.at[pl.ds(base, PER_W)], idx_s)
    pltpu.sync_copy(dst_h.at[pl.ds(base, PER_W)], idx_d)

    sbufs = [s0, s1, s2, s3]
    dbufs = [d0, d1, d2, d3]
    fts = [f0, f1, f2, f3]

    @pl.loop(0, N_CHUNKS)
    def _chunk(ch):
        off = ch * CHUNK
        copies = []
        for q in range(4):
            copies.append(pltpu.async_copy(
                fts[q].at[idx_s.at[pl.ds(off, CHUNK)]], sbufs[q], sem))
            copies.append(pltpu.async_copy(
                fts[q].at[idx_d.at[pl.ds(off, CHUNK)]], dbufs[q], sem))
        for cp in copies:
            cp.wait()

        for g in range(N_GROUPS):
            items = lax.iota(jnp.int32, L) + g * L
            accs = tuple(jnp.zeros((L,), jnp.float32) for _ in range(9))
            for (k0, k1, qs, cb) in SEGS:
                def kstep(k, carry, qs=qs, cb=cb):
                    accs = carry[:9]
                    cols = carry[9:]
                    sv = [plsc.load_gather(sbufs[qs[t]], [items, cols[t]])
                          for t in range(3)]
                    dv = [plsc.load_gather(dbufs[qs[t]], [items, cols[t]])
                          for t in range(3)]
                    new = tuple(accs[a * 3 + b] + sv[a] * dv[b]
                                for a in range(3) for b in range(3))
                    return new + tuple(c + 1 for c in cols)
                cols0 = tuple(jnp.full((L,), k0 + cb[t], jnp.int32)
                              for t in range(3))
                carry = lax.fori_loop(k0, k1, kstep, accs + cols0, unroll=2)
                accs = carry[:9]

            flat = items * OUT_DIM
            gsv = [plsc.load_gather(s3, [items, jnp.full((L,), G_COL + j,
                                                         jnp.int32)])
                   for j in range(6)]
            gdv = [plsc.load_gather(d3, [items, jnp.full((L,), G_COL + j,
                                                         jnp.int32)])
                   for j in range(6)]
            for i in range(3):
                for j in range(3):
                    pair = G_PAIR[(min(i, j), max(i, j))]
                    plsc.store_scatter(rfb, [flat + (i * 6 + j)], gsv[pair])
                    plsc.store_scatter(rfb, [flat + ((3 + i) * 6 + 3 + j)],
                                       gdv[pair])
                    v = accs[i * 3 + j]
                    plsc.store_scatter(rfb, [flat + (i * 6 + 3 + j)], v)
                    plsc.store_scatter(rfb, [flat + ((3 + j) * 6 + i)], v)

        pltpu.sync_copy(rfb, rf_h.at[pl.ds((base + off) * OUT_DIM,
                                           CHUNK * OUT_DIM)])


_sc_gram = functools.partial(
    pl.kernel,
    out_type=jax.ShapeDtypeStruct((BP * OUT_DIM,), jnp.float32),
    mesh=plsc.VectorSubcoreMesh(core_axis_name="c", subcore_axis_name="s"),
    scratch_types=[
        pltpu.VMEM((PER_W,), jnp.int32),
        pltpu.VMEM((PER_W,), jnp.int32),
    ] + [pltpu.VMEM((CHUNK, 128), jnp.float32) for _ in range(8)] + [
        pltpu.VMEM((CHUNK * OUT_DIM,), jnp.float32),
        pltpu.SemaphoreType.DMA,
    ],
    compiler_params=pltpu.CompilerParams(use_tc_tiling_on_sc=False,
                                         needs_layout_passes=False),
)(_sc_body)


BT = 2000


def _mlp_body(rf_ref, w1t_ref, b1_ref, w2t_ref, b2_ref, out_ref):
    x = jnp.log1p(jnp.maximum(rf_ref[...], 0.0))
    h = jnp.dot(x, w1t_ref[...], preferred_element_type=jnp.float32,
                precision=lax.Precision.HIGHEST)
    h = jnp.maximum(h + b1_ref[...], 0.0)
    o = jnp.dot(h, w2t_ref[...], preferred_element_type=jnp.float32,
                precision=lax.Precision.HIGHEST)
    out_ref[...] = o + b2_ref[...]


def _mlp(rf, w1t, b1, w2t, b2):
    grid = (B // BT,)
    return pl.pallas_call(
        _mlp_body,
        grid=grid,
        in_specs=[
            pl.BlockSpec((BT, OUT_DIM), lambda i: (i, 0)),
            pl.BlockSpec((OUT_DIM, HID), lambda i: (0, 0)),
            pl.BlockSpec((1, HID), lambda i: (0, 0)),
            pl.BlockSpec((HID, OUT_DIM), lambda i: (0, 0)),
            pl.BlockSpec((1, OUT_DIM), lambda i: (0, 0)),
        ],
        out_specs=pl.BlockSpec((BT, OUT_DIM), lambda i: (i, 0)),
        out_shape=jax.ShapeDtypeStruct((B, OUT_DIM), jnp.float32),
    )(rf, w1t, b1, w2t, b2)


def kernel(src, dst, p0, p1, p2, w1, b1, w2, b2):
    ft0, ft1, ft2, ft3 = _build_ft(p0, p1, p2)
    pad = BP - B
    src_p = jnp.concatenate([src.astype(jnp.int32),
                             jnp.zeros((pad,), jnp.int32)])
    dst_p = jnp.concatenate([dst.astype(jnp.int32),
                             jnp.zeros((pad,), jnp.int32)])
    rf1 = _sc_gram(src_p, dst_p, ft0, ft1, ft2, ft3)
    rf = rf1.reshape(BP, OUT_DIM)
    return _mlp(rf, w1.T, b1.reshape(1, HID), w2.T, b2.reshape(1, OUT_DIM))


# speedup vs baseline: 2.0881x; 1.0502x over previous
import functools

import jax
import jax.numpy as jnp
from jax import lax
from jax.experimental import pallas as pl
from jax.experimental.pallas import tpu as pltpu
from jax.experimental.pallas import tpu_sc as plsc

NUM_NODES = 100000
DIM = 150
OUT_DIM = 36
HID = 144

NC = 2
NS = 16
L = 16
NW = NC * NS

B = 100000
PER_W = 3136
BP = NW * PER_W
CHUNK = 32
N_CHUNKS = PER_W // CHUNK
N_GROUPS = CHUNK // L

SEGS = [
    (0, 84, (0, 1, 2), (0, 22, 44)),
    (84, 106, (0, 1, 3), (0, 22, -84)),
    (106, 128, (0, 2, 3), (0, -106, -84)),
    (128, 150, (1, 2, 3), (-128, -106, -84)),
]
G_COL = 66
G_PAIR = {(0, 0): 0, (0, 1): 1, (0, 2): 2, (1, 1): 3, (1, 2): 4, (2, 2): 5}


RN = 2000


def _ft_body(p0r, p1r, p2r, f0r, f1r, f2r, f3r):
    a0, a1, a2 = p0r[...], p1r[...], p2r[...]
    gs = []
    for (x, y) in [(a0, a0), (a0, a1), (a0, a2),
                   (a1, a1), (a1, a2), (a2, a2)]:
        gs.append(jnp.sum(x * y, axis=1, keepdims=True))
    f0r[...] = a0[:, :128]
    f1r[...] = jnp.concatenate([a0[:, 128:], a1[:, :106]], axis=1)
    f2r[...] = jnp.concatenate([a1[:, 106:], a2[:, :84]], axis=1)
    f3r[...] = jnp.concatenate(
        [a2[:, 84:]] + gs + [jnp.zeros((RN, 56), jnp.float32)], axis=1)


def _build_ft(p0, p1, p2):
    grid = (NUM_NODES // RN,)
    ft_shape = jax.ShapeDtypeStruct((NUM_NODES, 128), jnp.float32)
    return pl.pallas_call(
        _ft_body,
        grid=grid,
        in_specs=[pl.BlockSpec((RN, DIM), lambda i: (i, 0))] * 3,
        out_specs=[pl.BlockSpec((RN, 128), lambda i: (i, 0))] * 4,
        out_shape=[ft_shape] * 4,
    )(p0, p1, p2)


def _sc_body(src_h, dst_h, f0, f1, f2, f3, rf_h,
             idx_s, idx_d,
             s0a, s1a, s2a, s3a, d0a, d1a, d2a, d3a,
             s0b, s1b, s2b, s3b, d0b, d1b, d2b, d3b,
             rfb, sem_a, sem_b):
    cid = lax.axis_index("c")
    sid = lax.axis_index("s")
    wid = sid * NC + cid
    base = wid * PER_W

    pltpu.sync_copy(src_h.at[pl.ds(base, PER_W)], idx_s)
    pltpu.sync_copy(dst_h.at[pl.ds(base, PER_W)], idx_d)

    fts = [f0, f1, f2, f3]
    set_a = ([s0a, s1a, s2a, s3a], [d0a, d1a, d2a, d3a], sem_a)
    set_b = ([s0b, s1b, s2b, s3b], [d0b, d1b, d2b, d3b], sem_b)

    def fire(ch, bufset):
        sbufs, dbufs, sem = bufset
        off = ch * CHUNK
        for q in range(4):
            pltpu.async_copy(fts[q].at[idx_s.at[pl.ds(off, CHUNK)]],
                             sbufs[q], sem)
            pltpu.async_copy(fts[q].at[idx_d.at[pl.ds(off, CHUNK)]],
                             dbufs[q], sem)

    def drain(bufset):
        sbufs, dbufs, sem = bufset
        for q in range(4):
            pltpu.make_async_copy(fts[q].at[idx_s.at[pl.ds(0, CHUNK)]],
                                  sbufs[q], sem).wait()
            pltpu.make_async_copy(fts[q].at[idx_d.at[pl.ds(0, CHUNK)]],
                                  dbufs[q], sem).wait()

    def compute(ch, bufset):
        sbufs, dbufs, _ = bufset
        off = ch * CHUNK
        for g in range(N_GROUPS):
            items = lax.iota(jnp.int32, L) + g * L
            accs = tuple(jnp.zeros((L,), jnp.float32) for _ in range(9))
            for (k0, k1, qs, cb) in SEGS:
                def kstep(k, carry, qs=qs, cb=cb):
                    accs = carry[:9]
                    cols = carry[9:]
                    sv = [plsc.load_gather(sbufs[qs[t]], [items, cols[t]])
                          for t in range(3)]
                    dv = [plsc.load_gather(dbufs[qs[t]], [items, cols[t]])
                          for t in range(3)]
                    new = tuple(accs[a * 3 + b] + sv[a] * dv[b]
                                for a in range(3) for b in range(3))
                    return new + tuple(c + 1 for c in cols)
                cols0 = tuple(jnp.full((L,), k0 + cb[t], jnp.int32)
                              for t in range(3))
                carry = lax.fori_loop(k0, k1, kstep, accs + cols0, unroll=4)
                accs = carry[:9]

            flat = items * OUT_DIM
            gsv = [plsc.load_gather(sbufs[3],
                                    [items, jnp.full((L,), G_COL + j,
                                                     jnp.int32)])
                   for j in range(6)]
            gdv = [plsc.load_gather(dbufs[3],
                                    [items, jnp.full((L,), G_COL + j,
                                                     jnp.int32)])
                   for j in range(6)]
            for i in range(3):
                for j in range(3):
                    pair = G_PAIR[(min(i, j), max(i, j))]
                    plsc.store_scatter(rfb, [flat + (i * 6 + j)], gsv[pair])
                    plsc.store_scatter(rfb, [flat + ((3 + i) * 6 + 3 + j)],
                                       gdv[pair])
                    v = accs[i * 3 + j]
                    plsc.store_scatter(rfb, [flat + (i * 6 + 3 + j)], v)
                    plsc.store_scatter(rfb, [flat + ((3 + j) * 6 + i)], v)

        pltpu.sync_copy(rfb, rf_h.at[pl.ds((base + off) * OUT_DIM,
                                           CHUNK * OUT_DIM)])

    fire(0, set_a)

    @pl.loop(0, N_CHUNKS, step=2)
    def _chunk(ch):
        fire(ch + 1, set_b)
        drain(set_a)
        compute(ch, set_a)

        @pl.when(ch + 2 < N_CHUNKS)
        def _():
            fire(ch + 2, set_a)
        drain(set_b)
        compute(ch + 1, set_b)


_sc_gram = functools.partial(
    pl.kernel,
    out_type=jax.ShapeDtypeStruct((BP * OUT_DIM,), jnp.float32),
    mesh=plsc.VectorSubcoreMesh(core_axis_name="c", subcore_axis_name="s"),
    scratch_types=[
        pltpu.VMEM((PER_W,), jnp.int32),
        pltpu.VMEM((PER_W,), jnp.int32),
    ] + [pltpu.VMEM((CHUNK, 128), jnp.float32) for _ in range(16)] + [
        pltpu.VMEM((CHUNK * OUT_DIM,), jnp.float32),
        pltpu.SemaphoreType.DMA,
        pltpu.SemaphoreType.DMA,
    ],
    compiler_params=pltpu.CompilerParams(use_tc_tiling_on_sc=False,
                                         needs_layout_passes=False),
)(_sc_body)


BT = 2000


def _mlp_body(rf_ref, w1t_ref, b1_ref, w2t_ref, b2_ref, out_ref):
    x = jnp.log1p(jnp.maximum(rf_ref[...], 0.0))
    h = jnp.dot(x, w1t_ref[...], preferred_element_type=jnp.float32,
                precision=lax.Precision.HIGHEST)
    h = jnp.maximum(h + b1_ref[...], 0.0)
    o = jnp.dot(h, w2t_ref[...], preferred_element_type=jnp.float32,
                precision=lax.Precision.HIGHEST)
    out_ref[...] = o + b2_ref[...]


def _mlp(rf, w1t, b1, w2t, b2):
    grid = (B // BT,)
    return pl.pallas_call(
        _mlp_body,
        grid=grid,
        in_specs=[
            pl.BlockSpec((BT, OUT_DIM), lambda i: (i, 0)),
            pl.BlockSpec((OUT_DIM, HID), lambda i: (0, 0)),
            pl.BlockSpec((1, HID), lambda i: (0, 0)),
            pl.BlockSpec((HID, OUT_DIM), lambda i: (0, 0)),
            pl.BlockSpec((1, OUT_DIM), lambda i: (0, 0)),
        ],
        out_specs=pl.BlockSpec((BT, OUT_DIM), lambda i: (i, 0)),
        out_shape=jax.ShapeDtypeStruct((B, OUT_DIM), jnp.float32),
    )(rf, w1t, b1, w2t, b2)


def kernel(src, dst, p0, p1, p2, w1, b1, w2, b2):
    ft0, ft1, ft2, ft3 = _build_ft(p0, p1, p2)
    pad = BP - B
    src_p = jnp.concatenate([src.astype(jnp.int32),
                             jnp.zeros((pad,), jnp.int32)])
    dst_p = jnp.concatenate([dst.astype(jnp.int32),
                             jnp.zeros((pad,), jnp.int32)])
    rf1 = _sc_gram(src_p, dst_p, ft0, ft1, ft2, ft3)
    rf = rf1.reshape(BP, OUT_DIM)
    return _mlp(rf, w1.T, b1.reshape(1, HID), w2.T, b2.reshape(1, OUT_DIM))


# speedup vs baseline: 4.5901x; 2.1982x over previous
import functools

import jax
import jax.numpy as jnp
from jax import lax
from jax.experimental import pallas as pl
from jax.experimental.pallas import tpu as pltpu
from jax.experimental.pallas import tpu_sc as plsc

NUM_NODES = 100000
DIM = 150
OUT_DIM = 36
RF_PITCH = 37
HID = 144

NC = 2
NS = 16
L = 16
NW = NC * NS

B = 100000
PER_W = 3136
BP = NW * PER_W
CHUNK = 32
N_CHUNKS = PER_W // CHUNK
N_GROUPS = CHUNK // L

G_COL = 66
G_PAIR = {(0, 0): 0, (0, 1): 1, (0, 2): 2, (1, 1): 3, (1, 2): 4, (2, 2): 5}


RN = 2000


def _ft_body(p0r, p1r, p2r, fr):
    a0, a1, a2 = p0r[...], p1r[...], p2r[...]
    gs = []
    for (x, y) in [(a0, a0), (a0, a1), (a0, a2),
                   (a1, a1), (a1, a2), (a2, a2)]:
        gs.append(jnp.sum(x * y, axis=1, keepdims=True))
    part0 = a0[:, :128]
    part1 = jnp.concatenate([a0[:, 128:], a1[:, :106]], axis=1)
    part2 = jnp.concatenate([a1[:, 106:], a2[:, :84]], axis=1)
    part3 = jnp.concatenate(
        [a2[:, 84:]] + gs + [jnp.zeros((RN, 56), jnp.float32)], axis=1)
    fr[...] = jnp.concatenate([part0, part1, part2, part3], axis=0)


def _build_ft(p0, p1, p2):
    grid = (NUM_NODES // RN,)
    return pl.pallas_call(
        _ft_body,
        grid=grid,
        in_specs=[pl.BlockSpec((RN, DIM), lambda i: (i, 0))] * 3,
        out_specs=pl.BlockSpec((4 * RN, 128), lambda i: (i, 0)),
        out_shape=jax.ShapeDtypeStruct((4 * NUM_NODES, 128), jnp.float32),
    )(p0, p1, p2)


def _sc_body(src_h, dst_h, ft, rf_h,
             idx_s, idx_d, sa, da, sb, db, rfb, sem_a, sem_b):
    cid = lax.axis_index("c")
    sid = lax.axis_index("s")
    wid = sid * NC + cid
    base = wid * PER_W

    pltpu.sync_copy(src_h.at[pl.ds(base * 4, PER_W * 4)], idx_s)
    pltpu.sync_copy(dst_h.at[pl.ds(base * 4, PER_W * 4)], idx_d)

    set_a = (sa, da, sem_a)
    set_b = (sb, db, sem_b)

    def fire(ch, bufset):
        sbuf, dbuf, sem = bufset
        off = ch * CHUNK * 4
        pltpu.async_copy(ft.at[idx_s.at[pl.ds(off, CHUNK * 4)]], sbuf, sem)
        pltpu.async_copy(ft.at[idx_d.at[pl.ds(off, CHUNK * 4)]], dbuf, sem)

    def drain(bufset):
        sbuf, dbuf, sem = bufset
        pltpu.make_async_copy(ft.at[idx_s.at[pl.ds(0, CHUNK * 4)]],
                              sbuf, sem).wait()
        pltpu.make_async_copy(ft.at[idx_d.at[pl.ds(0, CHUNK * 4)]],
                              dbuf, sem).wait()

    zero16 = jnp.zeros((L,), jnp.int32)

    def compute(ch, bufset):
        sbuf, dbuf, _ = bufset
        off = ch * CHUNK
        for g in range(N_GROUPS):
            items = lax.iota(jnp.int32, L) + g * L
            items512 = items * 512
            accs = tuple(jnp.zeros((L,), jnp.float32) for _ in range(9))
            k0 = lax.iota(jnp.int32, L) * 9

            def kstep(_, carry):
                accs = carry[:9]
                k = carry[9]
                a0 = items512 + k
                a1 = a0 + 150
                a2 = a0 + 300
                sv = [plsc.load_gather(sbuf, [zero16, a])
                      for a in (a0, a1, a2)]
                dv = [plsc.load_gather(dbuf, [zero16, a])
                      for a in (a0, a1, a2)]
                new = tuple(accs[a * 3 + b] + sv[a] * dv[b]
                            for a in range(3) for b in range(3))
                kn = k + 1
                kn = jnp.where(kn == DIM, 0, kn)
                return new + (kn,)

            carry = lax.fori_loop(0, DIM, kstep, accs + (k0,), unroll=5)
            accs = carry[:9]

            flat = items * RF_PITCH
            gsv = [plsc.load_gather(sbuf, [zero16,
                                           items512 + (384 + G_COL + j)])
                   for j in range(6)]
            gdv = [plsc.load_gather(dbuf, [zero16,
                                           items512 + (384 + G_COL + j)])
                   for j in range(6)]
            for i in range(3):
                for j in range(3):
                    pair = G_PAIR[(min(i, j), max(i, j))]
                    plsc.store_scatter(rfb, [flat + (i * 6 + j)], gsv[pair])
                    plsc.store_scatter(rfb, [flat + ((3 + i) * 6 + 3 + j)],
                                       gdv[pair])
                    v = accs[i * 3 + j]
                    plsc.store_scatter(rfb, [flat + (i * 6 + 3 + j)], v)
                    plsc.store_scatter(rfb, [flat + ((3 + j) * 6 + i)], v)

        pltpu.sync_copy(rfb, rf_h.at[pl.ds((base + off) * RF_PITCH,
                                           CHUNK * RF_PITCH)])

    fire(0, set_a)

    @pl.loop(0, N_CHUNKS, step=2)
    def _chunk(ch):
        fire(ch + 1, set_b)
        drain(set_a)
        compute(ch, set_a)

        @pl.when(ch + 2 < N_CHUNKS)
        def _():
            fire(ch + 2, set_a)
        drain(set_b)
        compute(ch + 1, set_b)


_sc_gram = functools.partial(
    pl.kernel,
    out_type=jax.ShapeDtypeStruct((BP * RF_PITCH,), jnp.float32),
    mesh=plsc.VectorSubcoreMesh(core_axis_name="c", subcore_axis_name="s"),
    scratch_types=[
        pltpu.VMEM((PER_W * 4,), jnp.int32),
        pltpu.VMEM((PER_W * 4,), jnp.int32),
    ] + [pltpu.VMEM((CHUNK * 4, 128), jnp.float32) for _ in range(4)] + [
        pltpu.VMEM((CHUNK * RF_PITCH,), jnp.float32),
        pltpu.SemaphoreType.DMA,
        pltpu.SemaphoreType.DMA,
    ],
    compiler_params=pltpu.CompilerParams(use_tc_tiling_on_sc=False,
                                         needs_layout_passes=False,
                                         disable_bounds_checks=True),
)(_sc_body)


BT = 2000


def _mlp_body(rf_ref, w1t_ref, b1_ref, w2t_ref, b2_ref, out_ref):
    x = jnp.log1p(jnp.maximum(rf_ref[:, :OUT_DIM], 0.0))
    h = jnp.dot(x, w1t_ref[...], preferred_element_type=jnp.float32)
    h = jnp.maximum(h + b1_ref[...], 0.0)
    o = jnp.dot(h, w2t_ref[...], preferred_element_type=jnp.float32)
    out_ref[...] = o + b2_ref[...]


def _mlp(rf, w1t, b1, w2t, b2):
    grid = (B // BT,)
    return pl.pallas_call(
        _mlp_body,
        grid=grid,
        in_specs=[
            pl.BlockSpec((BT, RF_PITCH), lambda i: (i, 0)),
            pl.BlockSpec((OUT_DIM, HID), lambda i: (0, 0)),
            pl.BlockSpec((1, HID), lambda i: (0, 0)),
            pl.BlockSpec((HID, OUT_DIM), lambda i: (0, 0)),
            pl.BlockSpec((1, OUT_DIM), lambda i: (0, 0)),
        ],
        out_specs=pl.BlockSpec((BT, OUT_DIM), lambda i: (i, 0)),
        out_shape=jax.ShapeDtypeStruct((B, OUT_DIM), jnp.float32),
    )(rf, w1t, b1, w2t, b2)


def kernel(src, dst, p0, p1, p2, w1, b1, w2, b2):
    ft = _build_ft(p0, p1, p2)
    pad = BP - B
    src_p = jnp.concatenate([src.astype(jnp.int32),
                             jnp.zeros((pad,), jnp.int32)])
    dst_p = jnp.concatenate([dst.astype(jnp.int32),
                             jnp.zeros((pad,), jnp.int32)])
    qoff = jnp.arange(4, dtype=jnp.int32) * RN
    def expand(n):
        r = (n // RN) * (4 * RN) + (n % RN)
        return (r[:, None] + qoff[None, :]).reshape(-1)
    rf1 = _sc_gram(expand(src_p), expand(dst_p), ft)
    rf = rf1.reshape(BP, RF_PITCH)
    return _mlp(rf, w1.T, b1.reshape(1, HID), w2.T, b2.reshape(1, OUT_DIM))
